# baseline (speedup 1.0000x reference)
import dataclasses
import functools

import jax
import jax.numpy as jnp
from jax import lax
from jax.experimental import pallas as pl
from jax.experimental.pallas import tpu as pltpu
from jax.experimental.pallas import tpu_sc as plsc

T = 2048
C = 768
DFF = 3072
E = 8
G = 256
NB = 24
NSLOT = NB * G
NP = 2 * T
CCH = 768
NCH = DFF // CCH


def _router_body(x_ref, wr_ref, eb_ref, sel_ref, w_ref):
    x = x_ref[...]
    wr = wr_ref[...]
    logits = lax.dot_general(
        x, wr, (((1,), (1,)), ((), ())), preferred_element_type=jnp.float32)
    logits = logits + eb_ref[...]
    m1 = jnp.max(logits, axis=1, keepdims=True)
    iota = lax.broadcasted_iota(jnp.int32, logits.shape, 1)
    a1 = jnp.min(jnp.where(logits == m1, iota, E), axis=1, keepdims=True)
    lm = jnp.where(iota == a1, -jnp.inf, logits)
    m2 = jnp.max(lm, axis=1, keepdims=True)
    a2 = jnp.min(jnp.where(lm == m2, iota, E), axis=1, keepdims=True)
    e2 = jnp.exp(m2 - m1)
    den = 1.0 + e2
    rb = x.shape[0]
    sel_ref[0:1, :] = a1.reshape(1, rb)
    sel_ref[1:2, :] = a2.reshape(1, rb)
    w_ref[0:1, :] = (1.0 / den).reshape(1, rb)
    w_ref[1:2, :] = (e2 / den).reshape(1, rb)


def _router(x_flat, Wr, expert_bias):
    RB = 256
    return pl.pallas_call(
        _router_body,
        grid=(T // RB,),
        in_specs=[
            pl.BlockSpec((RB, C), lambda i: (i, 0)),
            pl.BlockSpec((E, C), lambda i: (0, 0)),
            pl.BlockSpec((1, E), lambda i: (0, 0)),
        ],
        out_specs=[
            pl.BlockSpec((2, RB), lambda i: (0, i)),
            pl.BlockSpec((2, RB), lambda i: (0, i)),
        ],
        out_shape=[
            jax.ShapeDtypeStruct((2, T), jnp.int32),
            jax.ShapeDtypeStruct((2, T), jnp.float32),
        ],
    )(x_flat, Wr, expert_bias.reshape(1, E))


_VMESH = dict(core_axis_name="c", subcore_axis_name="s")

_SC_PARAMS = pltpu.CompilerParams()
if "needs_layout_passes" in pltpu.CompilerParams.__dataclass_fields__:
    _SC_PARAMS = dataclasses.replace(_SC_PARAMS, needs_layout_passes=False)


def _gath16(arr, idx):
    return lax.gather(
        arr, idx.reshape(16, 1),
        lax.GatherDimensionNumbers(
            offset_dims=(), collapsed_slice_dims=(0,), start_index_map=(0,)),
        (1,), mode=lax.GatherScatterMode.PROMISE_IN_BOUNDS)


def _build_body(sel_hbm, w_hbm, order_hbm, gate_hbm, inv_hbm, meta_hbm,
                selv, wv, orderv, gatev, invv, metav, cntv,
                basev_ref, sem):
    wid = lax.axis_index("s") * 2 + lax.axis_index("c")

    @pl.when(wid == 0)
    def _():
        pltpu.async_copy(sel_hbm, selv, sem).wait()
        pltpu.async_copy(w_hbm, wv, sem).wait()

        zi = jnp.zeros((16,), jnp.int32)
        zf = jnp.zeros((16,), jnp.float32)
        it16 = lax.iota(jnp.int32, 16)
        prv = jnp.maximum(it16 - 1, 0)
        nxt = jnp.minimum(it16 + 1, 15)

        @pl.loop(0, NSLOT // 16)
        def _(i):
            orderv[pl.ds(i * 16, 16)] = zi
            gatev[pl.ds(i * 16, 16)] = zf

        cntv[...] = zi

        @pl.loop(0, NP // 16)
        def _(i):
            sv = selv[pl.ds(i * 16, 16)]
            sk, _unused = plsc.sort_key_val(sv, it16)
            newseg = jnp.logical_or(it16 == 0, sk != _gath16(sk, prv))
            segstart = plsc.cummax(jnp.where(newseg, it16, 0))
            rank = it16 - segstart
            segend = jnp.logical_or(it16 == 15, sk != _gath16(sk, nxt))
            plsc.addupdate_scatter(cntv, [sk], rank + 1, mask=segend)

        cnt = cntv[...]
        nbe = jnp.right_shift(cnt + (G - 1), 8)
        sizes = nbe * G
        offs_vec = plsc.cumsum(sizes) - sizes
        bs_vec = jnp.right_shift(offs_vec, 8)
        basev_ref[...] = offs_vec
        nb = jnp.sum(nbe)

        for v in range(2):
            bvec = it16 + v * 16
            bcl = jnp.minimum(bvec, nb - 1)
            cntacc = zi
            for e in range(E):
                bs_e = jnp.sum(jnp.where(it16 == e, bs_vec, 0))
                cntacc = cntacc + (bcl >= bs_e).astype(jnp.int32)
            metav[pl.ds(v * 16, 16)] = cntacc - 1
        metav[pl.ds(32, 16)] = jnp.full((16,), 0, jnp.int32) + nb

        @pl.loop(0, NP // 16)
        def _(i):
            sv = selv[pl.ds(i * 16, 16)]
            gv = wv[pl.ds(i * 16, 16)]
            sk, sv2 = plsc.sort_key_val(sv, it16)
            newseg = jnp.logical_or(it16 == 0, sk != _gath16(sk, prv))
            segstart = plsc.cummax(jnp.where(newseg, it16, 0))
            rank = it16 - segstart
            segend = jnp.logical_or(it16 == 15, sk != _gath16(sk, nxt))
            pos = plsc.load_gather(basev_ref, [sk]) + rank
            plsc.store_scatter(basev_ref, [sk], pos + 1, mask=segend)
            tok = (i * 16 + sv2) & (T - 1)
            plsc.store_scatter(orderv, [pos], tok)
            plsc.store_scatter(gatev, [pos], _gath16(gv, sv2))
            plsc.store_scatter(invv, [i * 16 + sv2], pos)

        pltpu.async_copy(orderv, order_hbm, sem).wait()
        pltpu.async_copy(gatev, gate_hbm, sem).wait()
        pltpu.async_copy(invv, inv_hbm, sem).wait()
        pltpu.async_copy(metav, meta_hbm, sem).wait()


def _build(sel_flat, w_flat):
    return pl.kernel(
        _build_body,
        out_type=(
            jax.ShapeDtypeStruct((NSLOT,), jnp.int32),
            jax.ShapeDtypeStruct((NSLOT,), jnp.float32),
            jax.ShapeDtypeStruct((NP,), jnp.int32),
            jax.ShapeDtypeStruct((48,), jnp.int32),
        ),
        mesh=plsc.VectorSubcoreMesh(**_VMESH),
        scratch_types=[
            pltpu.VMEM((NP,), jnp.int32),
            pltpu.VMEM((NP,), jnp.float32),
            pltpu.VMEM((NSLOT,), jnp.int32),
            pltpu.VMEM((NSLOT,), jnp.float32),
            pltpu.VMEM((NP,), jnp.int32),
            pltpu.VMEM((48,), jnp.int32),
            pltpu.VMEM((16,), jnp.int32),
            pltpu.VMEM((16,), jnp.int32),
            pltpu.SemaphoreType.DMA,
        ],
        compiler_params=_SC_PARAMS,
    )(sel_flat, w_flat)


def _combine_body(inv_hbm, yg_hbm, sh_hbm, out_hbm, idx0, idx1, acc, buf,
                  sem, gsem):
    wid = lax.axis_index("s") * 2 + lax.axis_index("c")
    n = T // 32
    base = wid * n
    pltpu.async_copy(inv_hbm.at[pl.ds(base, n)], idx0, sem).wait()
    pltpu.async_copy(inv_hbm.at[pl.ds(T + base, n)], idx1, sem).wait()
    dsh = pltpu.async_copy(sh_hbm.at[pl.ds(base, n)], acc, sem)
    d0 = pltpu.async_copy(yg_hbm.at[idx0], buf, gsem)
    dsh.wait()
    d0.wait()

    @pl.loop(0, n)
    def _(r):
        for cc in range(C // 16):
            sl = pl.ds(cc * 16, 16)
            acc[r, sl] = acc[r, sl] + buf[r, sl]

    pltpu.async_copy(yg_hbm.at[idx1], buf, gsem).wait()

    @pl.loop(0, n)
    def _(r):
        for cc in range(C // 16):
            sl = pl.ds(cc * 16, 16)
            acc[r, sl] = acc[r, sl] + buf[r, sl]

    pltpu.async_copy(acc, out_hbm.at[pl.ds(base, n)], sem).wait()


def _combine(inv, yg, sh):
    n = T // 32
    return pl.kernel(
        _combine_body,
        out_type=jax.ShapeDtypeStruct((T, C), jnp.float32),
        mesh=plsc.VectorSubcoreMesh(**_VMESH),
        scratch_types=[
            pltpu.VMEM((n,), jnp.int32),
            pltpu.VMEM((n,), jnp.int32),
            pltpu.VMEM((n, C), jnp.float32),
            pltpu.VMEM((n, C), jnp.float32),
            pltpu.SemaphoreType.DMA,
            pltpu.SemaphoreType.DMA,
        ],
    )(inv, yg, sh)


def _shared_body(x_ref, w1_ref, b1_ref, w2_ref, b2_ref, out_ref, acc_ref):
    c = pl.program_id(0)

    @pl.when(c == 0)
    def _():
        acc_ref[...] = jnp.zeros_like(acc_ref)

    x = x_ref[...]
    w1 = w1_ref[...].astype(jnp.bfloat16)
    h = lax.dot_general(
        x, w1, (((1,), (1,)), ((), ())), preferred_element_type=jnp.float32)
    h = jnp.maximum(h + b1_ref[0], 0.0)
    hb = h.astype(jnp.bfloat16)
    w2 = w2_ref[...].astype(jnp.bfloat16)
    acc_ref[...] += lax.dot_general(
        hb, w2, (((1,), (1,)), ((), ())), preferred_element_type=jnp.float32)

    @pl.when(c == NCH - 1)
    def _():
        out_ref[...] = acc_ref[...] + b2_ref[...]


def _shared(x_bf, sW1, sb1, sW2, sb2):
    return pl.pallas_call(
        _shared_body,
        grid=(NCH,),
        in_specs=[
            pl.BlockSpec((T, C), lambda c: (0, 0)),
            pl.BlockSpec((CCH, C), lambda c: (c, 0)),
            pl.BlockSpec((1, 1, CCH), lambda c: (c, 0, 0)),
            pl.BlockSpec((C, CCH), lambda c: (0, c)),
            pl.BlockSpec((1, C), lambda c: (0, 0)),
        ],
        out_specs=pl.BlockSpec((T, C), lambda c: (0, 0)),
        out_shape=jax.ShapeDtypeStruct((T, C), jnp.float32),
        scratch_shapes=[pltpu.VMEM((T, C), jnp.float32)],
    )(x_bf, sW1.reshape(DFF, C), sb1.reshape(NCH, 1, CCH),
      sW2.reshape(C, DFF), sb2.reshape(1, C))


def _grouped_body(be_ref, nb_ref, x_ref, ord_ref, w1_ref, b1_ref, w2_ref,
                  b2_ref, g_ref, yg_ref, w1c_ref, w2c_ref):
    b = pl.program_id(0)

    @pl.when(b < nb_ref[0])
    def _():
        changed = jnp.logical_or(b == 0, be_ref[b] != be_ref[b - 1])

        @pl.when(changed)
        def _():
            w1c_ref[...] = w1_ref[0].astype(jnp.bfloat16)
            w2c_ref[...] = w2_ref[0].astype(jnp.bfloat16)

        ordc = ord_ref[0].reshape(-1, 1)
        tok = lax.broadcasted_iota(jnp.int32, (G, T), 1)
        P = (tok == ordc).astype(jnp.bfloat16)
        x = lax.dot_general(
            P, x_ref[...], (((1,), (0,)), ((), ())),
            preferred_element_type=jnp.float32).astype(jnp.bfloat16)
        h = lax.dot_general(
            x, w1c_ref[...], (((1,), (1,)), ((), ())),
            preferred_element_type=jnp.float32)
        h = jnp.maximum(h + b1_ref[0], 0.0)
        hb = h.astype(jnp.bfloat16)
        y = lax.dot_general(
            hb, w2c_ref[...], (((1,), (1,)), ((), ())),
            preferred_element_type=jnp.float32)
        y = y + b2_ref[0]
        g = g_ref[0].reshape(-1, 1)
        yg_ref[...] = g * y


def _grouped(be, nb, x_bf, order3, rW1, rb1, rW2, rb2, g3):
    grid_spec = pltpu.PrefetchScalarGridSpec(
        num_scalar_prefetch=2,
        grid=(NB,),
        in_specs=[
            pl.BlockSpec((T, C), lambda b, be, nb: (0, 0)),
            pl.BlockSpec((1, 1, G), lambda b, be, nb: (b, 0, 0)),
            pl.BlockSpec((1, DFF, C), lambda b, be, nb: (be[b], 0, 0)),
            pl.BlockSpec((1, 1, DFF), lambda b, be, nb: (be[b], 0, 0)),
            pl.BlockSpec((1, C, DFF), lambda b, be, nb: (be[b], 0, 0)),
            pl.BlockSpec((1, 1, C), lambda b, be, nb: (be[b], 0, 0)),
            pl.BlockSpec((1, 1, G), lambda b, be, nb: (b, 0, 0)),
        ],
        out_specs=pl.BlockSpec((G, C), lambda b, be, nb: (b, 0)),
        scratch_shapes=[pltpu.VMEM((DFF, C), jnp.bfloat16),
                        pltpu.VMEM((C, DFF), jnp.bfloat16)],
    )
    return pl.pallas_call(
        _grouped_body,
        grid_spec=grid_spec,
        out_shape=jax.ShapeDtypeStruct((NSLOT, C), jnp.float32),
        compiler_params=pltpu.CompilerParams(
            vmem_limit_bytes=62 * 1024 * 1024),
    )(be, nb, x_bf, order3, rW1, rb1.reshape(E, 1, DFF), rW2,
      rb2.reshape(E, 1, C), g3)


def kernel(x, sW1, sb1, sW2, sb2, rW1, rb1, rW2, rb2, Wr, expert_bias):
    B = x.shape[0]
    x_flat = x.reshape(T, C)
    x_bf = x_flat.astype(jnp.bfloat16)

    sel2, w2 = _router(x_flat, Wr, expert_bias)
    order, gatep, inv, meta = _build(sel2.reshape(NP), w2.reshape(NP))

    sh = _shared(x_bf, sW1, sb1, sW2, sb2)
    g3 = gatep.reshape(NB, 1, G)
    order3 = order.reshape(NB, 1, G)
    yg = _grouped(meta[:32], meta[32:33], x_bf, order3, rW1, rb1, rW2, rb2,
                  g3)
    out = _combine(inv, yg, sh)
    return out.reshape(B, T, C)


# speedup vs baseline: 1.0365x; 1.0365x over previous
import dataclasses
import functools

import jax
import jax.numpy as jnp
from jax import lax
from jax.experimental import pallas as pl
from jax.experimental.pallas import tpu as pltpu
from jax.experimental.pallas import tpu_sc as plsc

T = 2048
C = 768
DFF = 3072
E = 8
G = 256
NB = 24
NSLOT = NB * G
NP = 2 * T
CCH = 768
NCH = DFF // CCH


def _router_body(x_ref, wr_ref, eb_ref, sel_ref, w_ref):
    x = x_ref[...]
    wr = wr_ref[...]
    logits = lax.dot_general(
        x, wr, (((1,), (1,)), ((), ())), preferred_element_type=jnp.float32)
    logits = logits + eb_ref[...]
    m1 = jnp.max(logits, axis=1, keepdims=True)
    iota = lax.broadcasted_iota(jnp.int32, logits.shape, 1)
    a1 = jnp.min(jnp.where(logits == m1, iota, E), axis=1, keepdims=True)
    lm = jnp.where(iota == a1, -jnp.inf, logits)
    m2 = jnp.max(lm, axis=1, keepdims=True)
    a2 = jnp.min(jnp.where(lm == m2, iota, E), axis=1, keepdims=True)
    e2 = jnp.exp(m2 - m1)
    den = 1.0 + e2
    rb = x.shape[0]
    sel_ref[0:1, :] = a1.reshape(1, rb)
    sel_ref[1:2, :] = a2.reshape(1, rb)
    w_ref[0:1, :] = (1.0 / den).reshape(1, rb)
    w_ref[1:2, :] = (e2 / den).reshape(1, rb)


def _router(x_flat, Wr, expert_bias):
    RB = 256
    return pl.pallas_call(
        _router_body,
        grid=(T // RB,),
        in_specs=[
            pl.BlockSpec((RB, C), lambda i: (i, 0)),
            pl.BlockSpec((E, C), lambda i: (0, 0)),
            pl.BlockSpec((1, E), lambda i: (0, 0)),
        ],
        out_specs=[
            pl.BlockSpec((2, RB), lambda i: (0, i)),
            pl.BlockSpec((2, RB), lambda i: (0, i)),
        ],
        out_shape=[
            jax.ShapeDtypeStruct((2, T), jnp.int32),
            jax.ShapeDtypeStruct((2, T), jnp.float32),
        ],
    )(x_flat, Wr, expert_bias.reshape(1, E))


_VMESH = dict(core_axis_name="c", subcore_axis_name="s")

_SC_PARAMS = pltpu.CompilerParams()
if "needs_layout_passes" in pltpu.CompilerParams.__dataclass_fields__:
    _SC_PARAMS = dataclasses.replace(_SC_PARAMS, needs_layout_passes=False)


def _gath16(arr, idx):
    return lax.gather(
        arr, idx.reshape(16, 1),
        lax.GatherDimensionNumbers(
            offset_dims=(), collapsed_slice_dims=(0,), start_index_map=(0,)),
        (1,), mode=lax.GatherScatterMode.PROMISE_IN_BOUNDS)


def _build_body(sel_hbm, w_hbm, order_hbm, gate_hbm, inv_hbm, meta_hbm,
                selv, wv, orderv, gatev, invv, metav, cntv,
                basev_ref, sem):
    wid = lax.axis_index("s") * 2 + lax.axis_index("c")

    @pl.when(wid == 0)
    def _():
        pltpu.async_copy(sel_hbm, selv, sem).wait()
        pltpu.async_copy(w_hbm, wv, sem).wait()

        zi = jnp.zeros((16,), jnp.int32)
        zf = jnp.zeros((16,), jnp.float32)
        it16 = lax.iota(jnp.int32, 16)
        prv = jnp.maximum(it16 - 1, 0)
        nxt = jnp.minimum(it16 + 1, 15)

        @pl.loop(0, NSLOT // 16)
        def _(i):
            orderv[pl.ds(i * 16, 16)] = zi
            gatev[pl.ds(i * 16, 16)] = zf

        cntv[...] = zi

        @pl.loop(0, NP // 16)
        def _(i):
            sv = selv[pl.ds(i * 16, 16)]
            sk, _unused = plsc.sort_key_val(sv, it16)
            newseg = jnp.logical_or(it16 == 0, sk != _gath16(sk, prv))
            segstart = plsc.cummax(jnp.where(newseg, it16, 0))
            rank = it16 - segstart
            segend = jnp.logical_or(it16 == 15, sk != _gath16(sk, nxt))
            plsc.addupdate_scatter(cntv, [sk], rank + 1, mask=segend)

        cnt = cntv[...]
        nbe = jnp.right_shift(cnt + (G - 1), 8)
        sizes = nbe * G
        offs_vec = plsc.cumsum(sizes) - sizes
        bs_vec = jnp.right_shift(offs_vec, 8)
        basev_ref[...] = offs_vec
        nb = jnp.sum(nbe)

        for v in range(2):
            bvec = it16 + v * 16
            bcl = jnp.minimum(bvec, nb - 1)
            cntacc = zi
            for e in range(E):
                bs_e = jnp.sum(jnp.where(it16 == e, bs_vec, 0))
                cntacc = cntacc + (bcl >= bs_e).astype(jnp.int32)
            metav[pl.ds(v * 16, 16)] = cntacc - 1
        metav[pl.ds(32, 16)] = jnp.full((16,), 0, jnp.int32) + nb

        @pl.loop(0, NP // 16)
        def _(i):
            sv = selv[pl.ds(i * 16, 16)]
            gv = wv[pl.ds(i * 16, 16)]
            sk, sv2 = plsc.sort_key_val(sv, it16)
            newseg = jnp.logical_or(it16 == 0, sk != _gath16(sk, prv))
            segstart = plsc.cummax(jnp.where(newseg, it16, 0))
            rank = it16 - segstart
            segend = jnp.logical_or(it16 == 15, sk != _gath16(sk, nxt))
            pos = plsc.load_gather(basev_ref, [sk]) + rank
            plsc.store_scatter(basev_ref, [sk], pos + 1, mask=segend)
            tok = (i * 16 + sv2) & (T - 1)
            plsc.store_scatter(orderv, [pos], tok)
            plsc.store_scatter(gatev, [pos], _gath16(gv, sv2))
            plsc.store_scatter(invv, [i * 16 + sv2], pos)

        pltpu.async_copy(orderv, order_hbm, sem).wait()
        pltpu.async_copy(gatev, gate_hbm, sem).wait()
        pltpu.async_copy(invv, inv_hbm, sem).wait()
        pltpu.async_copy(metav, meta_hbm, sem).wait()


def _build(sel_flat, w_flat):
    return pl.kernel(
        _build_body,
        out_type=(
            jax.ShapeDtypeStruct((NSLOT,), jnp.int32),
            jax.ShapeDtypeStruct((NSLOT,), jnp.float32),
            jax.ShapeDtypeStruct((NP,), jnp.int32),
            jax.ShapeDtypeStruct((48,), jnp.int32),
        ),
        mesh=plsc.VectorSubcoreMesh(**_VMESH),
        scratch_types=[
            pltpu.VMEM((NP,), jnp.int32),
            pltpu.VMEM((NP,), jnp.float32),
            pltpu.VMEM((NSLOT,), jnp.int32),
            pltpu.VMEM((NSLOT,), jnp.float32),
            pltpu.VMEM((NP,), jnp.int32),
            pltpu.VMEM((48,), jnp.int32),
            pltpu.VMEM((16,), jnp.int32),
            pltpu.VMEM((16,), jnp.int32),
            pltpu.SemaphoreType.DMA,
        ],
        compiler_params=_SC_PARAMS,
    )(sel_flat, w_flat)


def _combine_body(inv_hbm, yg_hbm, sh_hbm, out_hbm, idx0, idx1, acc, buf,
                  sem, gsem):
    wid = lax.axis_index("s") * 2 + lax.axis_index("c")
    n = T // 32
    base = wid * n
    pltpu.async_copy(inv_hbm.at[pl.ds(base, n)], idx0, sem).wait()
    pltpu.async_copy(inv_hbm.at[pl.ds(T + base, n)], idx1, sem).wait()
    dsh = pltpu.async_copy(sh_hbm.at[pl.ds(base, n)], acc, sem)
    d0 = pltpu.async_copy(yg_hbm.at[idx0], buf, gsem)
    dsh.wait()
    d0.wait()

    @pl.loop(0, n)
    def _(r):
        for cc in range(C // 16):
            sl = pl.ds(cc * 16, 16)
            acc[r, sl] = acc[r, sl] + buf[r, sl]

    pltpu.async_copy(yg_hbm.at[idx1], buf, gsem).wait()

    @pl.loop(0, n)
    def _(r):
        for cc in range(C // 16):
            sl = pl.ds(cc * 16, 16)
            acc[r, sl] = acc[r, sl] + buf[r, sl]

    pltpu.async_copy(acc, out_hbm.at[pl.ds(base, n)], sem).wait()


def _combine(inv, yg, sh):
    n = T // 32
    return pl.kernel(
        _combine_body,
        out_type=jax.ShapeDtypeStruct((T, C), jnp.float32),
        mesh=plsc.VectorSubcoreMesh(**_VMESH),
        scratch_types=[
            pltpu.VMEM((n,), jnp.int32),
            pltpu.VMEM((n,), jnp.int32),
            pltpu.VMEM((n, C), jnp.float32),
            pltpu.VMEM((n, C), jnp.float32),
            pltpu.SemaphoreType.DMA,
            pltpu.SemaphoreType.DMA,
        ],
    )(inv, yg, sh)


def _shared_body(x_ref, w1_ref, b1_ref, w2_ref, b2_ref, out_ref, acc_ref):
    c = pl.program_id(0)

    @pl.when(c == 0)
    def _():
        acc_ref[...] = jnp.zeros_like(acc_ref)

    x = x_ref[...]
    w1 = w1_ref[...].astype(jnp.bfloat16)
    h = lax.dot_general(
        x, w1, (((1,), (1,)), ((), ())), preferred_element_type=jnp.float32)
    h = jnp.maximum(h + b1_ref[0], 0.0)
    hb = h.astype(jnp.bfloat16)
    w2 = w2_ref[...].astype(jnp.bfloat16)
    acc_ref[...] += lax.dot_general(
        hb, w2, (((1,), (1,)), ((), ())), preferred_element_type=jnp.float32)

    @pl.when(c == NCH - 1)
    def _():
        out_ref[...] = acc_ref[...] + b2_ref[...]


def _shared(x_bf, sW1, sb1, sW2, sb2):
    return pl.pallas_call(
        _shared_body,
        grid=(NCH,),
        in_specs=[
            pl.BlockSpec((T, C), lambda c: (0, 0)),
            pl.BlockSpec((CCH, C), lambda c: (c, 0)),
            pl.BlockSpec((1, 1, CCH), lambda c: (c, 0, 0)),
            pl.BlockSpec((C, CCH), lambda c: (0, c)),
            pl.BlockSpec((1, C), lambda c: (0, 0)),
        ],
        out_specs=pl.BlockSpec((T, C), lambda c: (0, 0)),
        out_shape=jax.ShapeDtypeStruct((T, C), jnp.float32),
        scratch_shapes=[pltpu.VMEM((T, C), jnp.float32)],
    )(x_bf, sW1.reshape(DFF, C), sb1.reshape(NCH, 1, CCH),
      sW2.reshape(C, DFF), sb2.reshape(1, C))


def _grouped_body(be_ref, nb_ref, x_ref, ord_ref, w1_ref, b1_ref, w2_ref,
                  b2_ref, g_ref, yg_ref):
    b = pl.program_id(0)

    @pl.when(b < nb_ref[0])
    def _():
        ordc = ord_ref[0].reshape(-1, 1)
        tok = lax.broadcasted_iota(jnp.int32, (G, T), 1)
        P = (tok == ordc).astype(jnp.bfloat16)
        x = lax.dot_general(
            P, x_ref[...], (((1,), (0,)), ((), ())),
            preferred_element_type=jnp.float32).astype(jnp.bfloat16)
        w1 = w1_ref[0].astype(jnp.bfloat16)
        h = lax.dot_general(
            x, w1, (((1,), (1,)), ((), ())), preferred_element_type=jnp.float32)
        h = jnp.maximum(h + b1_ref[0], 0.0)
        hb = h.astype(jnp.bfloat16)
        w2 = w2_ref[0].astype(jnp.bfloat16)
        y = lax.dot_general(
            hb, w2, (((1,), (1,)), ((), ())), preferred_element_type=jnp.float32)
        y = y + b2_ref[0]
        g = g_ref[0].reshape(-1, 1)
        yg_ref[...] = g * y


def _grouped(be, nb, x_bf, order3, rW1, rb1, rW2, rb2, g3):
    grid_spec = pltpu.PrefetchScalarGridSpec(
        num_scalar_prefetch=2,
        grid=(NB,),
        in_specs=[
            pl.BlockSpec((T, C), lambda b, be, nb: (0, 0)),
            pl.BlockSpec((1, 1, G), lambda b, be, nb: (b, 0, 0)),
            pl.BlockSpec((1, DFF, C), lambda b, be, nb: (be[b], 0, 0)),
            pl.BlockSpec((1, 1, DFF), lambda b, be, nb: (be[b], 0, 0)),
            pl.BlockSpec((1, C, DFF), lambda b, be, nb: (be[b], 0, 0)),
            pl.BlockSpec((1, 1, C), lambda b, be, nb: (be[b], 0, 0)),
            pl.BlockSpec((1, 1, G), lambda b, be, nb: (b, 0, 0)),
        ],
        out_specs=pl.BlockSpec((G, C), lambda b, be, nb: (b, 0)),
    )
    return pl.pallas_call(
        _grouped_body,
        grid_spec=grid_spec,
        out_shape=jax.ShapeDtypeStruct((NSLOT, C), jnp.float32),
        compiler_params=pltpu.CompilerParams(
            vmem_limit_bytes=60 * 1024 * 1024),
    )(be, nb, x_bf, order3, rW1, rb1.reshape(E, 1, DFF), rW2,
      rb2.reshape(E, 1, C), g3)


def kernel(x, sW1, sb1, sW2, sb2, rW1, rb1, rW2, rb2, Wr, expert_bias):
    B = x.shape[0]
    x_flat = x.reshape(T, C)
    x_bf = x_flat.astype(jnp.bfloat16)

    sel2, w2 = _router(x_flat, Wr, expert_bias)
    order, gatep, inv, meta = _build(sel2.reshape(NP), w2.reshape(NP))

    sh = _shared(x_bf, sW1, sb1, sW2, sb2)
    g3 = gatep.reshape(NB, 1, G)
    order3 = order.reshape(NB, 1, G)
    yg = _grouped(meta[:32], meta[32:33], x_bf, order3, rW1, rb1, rW2, rb2,
                  g3)
    out = _combine(inv, yg, sh)
    return out.reshape(B, T, C)
